# baseline (speedup 1.0000x reference)
import jax
import jax.numpy as jnp
from jax import lax
from jax.experimental import pallas as pl
from jax.experimental.pallas import tpu as pltpu
from jax.experimental.pallas import tpu_sc as plsc

GRID_H, GRID_W = 10, 11
NSLOT = GRID_H * GRID_W
NIDX = 112
NW = 32


def _sc_scatter(x_rows, base_idx, zeros_tail, B, C, T):
    nb = B // NW

    mesh = plsc.VectorSubcoreMesh(core_axis_name="c", subcore_axis_name="s")

    @pl.kernel(
        out_type=jax.ShapeDtypeStruct((B * NSLOT, T), jnp.float32),
        mesh=mesh,
        scratch_types=[
            pltpu.VMEM((NIDX, T), jnp.float32),
            pltpu.VMEM((NIDX, T), jnp.float32),
            pltpu.VMEM((NIDX,), jnp.int32),
            pltpu.VMEM((NIDX,), jnp.int32),
            pltpu.VMEM((NIDX,), jnp.int32),
            pltpu.SemaphoreType.DMA,
            pltpu.SemaphoreType.DMA,
            pltpu.SemaphoreType.DMA,
            pltpu.SemaphoreType.DMA,
        ],
    )
    def k(x_hbm, base_hbm, z_hbm, out_hbm,
          buf0, buf1, idx0, idx1, base_v,
          semL0, semL1, semS0, semS1):
        wid = lax.axis_index("s") * 2 + lax.axis_index("c")
        b_lo = wid * nb

        pltpu.sync_copy(base_hbm, base_v)
        pltpu.sync_copy(z_hbm, buf0.at[pl.ds(C, NIDX - C)])
        pltpu.sync_copy(z_hbm, buf1.at[pl.ds(C, NIDX - C)])

        slots = ((buf0, idx0, semL0, semS0), (buf1, idx1, semL1, semS1))

        def start_load(b, buf, semL):
            return pltpu.async_copy(
                x_hbm.at[pl.ds((b_lo + b) * C, C)],
                buf.at[pl.ds(0, C)], semL)

        def set_idx(b, idxr):
            off = (b_lo + b) * NSLOT
            for kk in range(NIDX // 16):
                sl = pl.ds(16 * kk, 16)
                idxr[sl] = base_v[sl] + off

        def start_scatter(buf, idxr, semS):
            return pltpu.async_copy(buf, out_hbm.at[idxr], semS)

        def wait_load(buf, semL):
            pltpu.make_async_copy(
                x_hbm.at[pl.ds(0, C)], buf.at[pl.ds(0, C)], semL).wait()

        def wait_scatter(buf, idxr, semS):
            pltpu.make_async_copy(buf, out_hbm.at[idxr], semS).wait()

        for j in range(2):
            buf, idxr, semL, semS = slots[j]
            start_load(j, buf, semL)
            set_idx(j, idxr)
            wait_load(buf, semL)
            start_scatter(buf, idxr, semS)

        @pl.loop(2, nb, step=2)
        def _(g):
            for j in range(2):
                buf, idxr, semL, semS = slots[j]
                b = g + j
                wait_scatter(buf, idxr, semS)
                start_load(b, buf, semL)
                set_idx(b, idxr)
                wait_load(buf, semL)
                start_scatter(buf, idxr, semS)

        for j in range(2):
            buf, idxr, _, semS = slots[j]
            wait_scatter(buf, idxr, semS)

    return k(x_rows, base_idx, zeros_tail)


def kernel(x, channel_transformation):
    B, C, T = x.shape
    ct = channel_transformation.astype(jnp.int32)

    slots = jnp.arange(NSLOT, dtype=jnp.int32)
    occupied = (slots[:, None] == ct[None, :]).any(axis=1)
    empty = jnp.where(~occupied, size=NSLOT - C, fill_value=0)[0]
    empty = empty.astype(jnp.int32)
    base_idx = jnp.concatenate([ct, empty, empty[-2:]])

    zeros_tail = jnp.zeros((NIDX - C, T), jnp.float32)
    x_rows = x.reshape(B * C, T)

    out_rows = _sc_scatter(x_rows, base_idx, zeros_tail, B, C, T)
    return out_rows.reshape(B, GRID_H, GRID_W, T)


# speedup vs baseline: 3.5318x; 3.5318x over previous
import jax
import jax.numpy as jnp
from jax import lax
from jax.experimental import pallas as pl
from jax.experimental.pallas import tpu as pltpu
from jax.experimental.pallas import tpu_sc as plsc

GRID_H, GRID_W = 10, 11
NSLOT = GRID_H * GRID_W
NW = 32
NRING = 4
CHUNK = 2


def _sc_scatter(x_rows, rel_d, rel_z, zeros_src, B, C, T):
    nb = B // NW
    nm = nb // CHUNK
    rows = CHUNK * C

    mesh = plsc.VectorSubcoreMesh(core_axis_name="c", subcore_axis_name="s")

    @pl.kernel(
        out_type=jax.ShapeDtypeStruct((NSLOT * B, T), jnp.float32),
        mesh=mesh,
        scratch_types=[
            pltpu.VMEM((rows, T), jnp.float32),
            pltpu.VMEM((rows, T), jnp.float32),
            pltpu.VMEM((rows, T), jnp.float32),
            pltpu.VMEM((rows, T), jnp.float32),
            pltpu.VMEM((rows, T), jnp.float32),
            pltpu.VMEM((nm, rows), jnp.int32),
            pltpu.VMEM((nm, rows), jnp.int32),
            pltpu.SemaphoreType.DMA,
            pltpu.SemaphoreType.DMA,
            pltpu.SemaphoreType.DMA,
            pltpu.SemaphoreType.DMA,
            pltpu.SemaphoreType.DMA,
            pltpu.SemaphoreType.DMA,
            pltpu.SemaphoreType.DMA,
            pltpu.SemaphoreType.DMA,
            pltpu.SemaphoreType.DMA,
        ],
    )
    def k(x_hbm, rel_d_hbm, rel_z_hbm, z_hbm, out_hbm,
          buf0, buf1, buf2, buf3, zbuf, didx, zidx,
          semL0, semL1, semL2, semL3,
          semD0, semD1, semD2, semD3, semZ):
        wid = lax.axis_index("s") * 2 + lax.axis_index("c")
        b_lo = wid * nb
        x_lo = b_lo * C

        bufs = (buf0, buf1, buf2, buf3)
        semL = (semL0, semL1, semL2, semL3)
        semD = (semD0, semD1, semD2, semD3)

        pltpu.sync_copy(z_hbm, zbuf)
        pltpu.sync_copy(rel_d_hbm, didx)
        pltpu.sync_copy(rel_z_hbm, zidx)

        @pl.loop(0, nm)
        def _(r):
            for kk in range(rows // 16):
                sl = pl.ds(16 * kk, 16)
                didx[r, sl] = didx[r, sl] + b_lo
                zidx[r, sl] = zidx[r, sl] + b_lo

        def start_load(m, j):
            return pltpu.async_copy(
                x_hbm.at[pl.ds(x_lo + m * rows, rows)], bufs[j], semL[j])

        def start_dscatter(m, j):
            return pltpu.async_copy(bufs[j], out_hbm.at[didx.at[m]], semD[j])

        def start_zscatter(m):
            return pltpu.async_copy(zbuf, out_hbm.at[zidx.at[m]], semZ)

        def wait_load(j):
            pltpu.make_async_copy(
                x_hbm.at[pl.ds(0, rows)], bufs[j], semL[j]).wait()

        def wait_dscatter(j):
            pltpu.make_async_copy(
                bufs[j], out_hbm.at[didx.at[0]], semD[j]).wait()

        def wait_zscatter():
            pltpu.make_async_copy(zbuf, out_hbm.at[zidx.at[0]], semZ).wait()

        for m in range(NRING):
            start_load(m, m)
            start_zscatter(m)
            wait_load(m)
            start_dscatter(m, m)

        @pl.loop(NRING, nm, step=NRING)
        def _(g):
            for j in range(NRING):
                m = g + j
                wait_dscatter(j)
                start_load(m, j)
                start_zscatter(m)
                wait_zscatter()
                wait_load(j)
                start_dscatter(m, j)

        for j in range(NRING):
            wait_dscatter(j)
            wait_zscatter()

    return k(x_rows, rel_d, rel_z, zeros_src)


def kernel(x, channel_transformation):
    B, C, T = x.shape
    ct = channel_transformation.astype(jnp.int32)

    slots = jnp.arange(NSLOT, dtype=jnp.int32)
    occupied = (slots[:, None] == ct[None, :]).any(axis=1)
    pz = jnp.where(~occupied, size=NSLOT - C, fill_value=0)[0].astype(jnp.int32)

    rows = CHUNK * C
    nm = (B // NW) // CHUNK
    mm = jnp.arange(nm, dtype=jnp.int32)[:, None]
    ii = jnp.arange(rows, dtype=jnp.int32)[None, :]
    rel_d = ct[ii % C] * B + CHUNK * mm + ii // C
    nz = CHUNK * (NSLOT - C)
    zi = jnp.where(ii < nz, ii, ii - (rows - nz))
    rel_z = pz[zi // CHUNK] * B + CHUNK * mm + zi % CHUNK

    zeros_src = jnp.zeros((rows, T), jnp.float32)
    x_rows = x.reshape(B * C, T)

    out_rows = _sc_scatter(x_rows, rel_d.astype(jnp.int32),
                           rel_z.astype(jnp.int32), zeros_src, B, C, T)
    return jnp.transpose(out_rows.reshape(GRID_H, GRID_W, B, T), (2, 0, 1, 3))


# speedup vs baseline: 3.6975x; 1.0469x over previous
import jax
import jax.numpy as jnp
from jax import lax
from jax.experimental import pallas as pl
from jax.experimental.pallas import tpu as pltpu
from jax.experimental.pallas import tpu_sc as plsc

GRID_H, GRID_W = 10, 11
NSLOT = GRID_H * GRID_W
NW = 32
NRING = 4
CHUNK = 2


def _sc_scatter(x_rows, tab_d, tab_z, zeros_src, B, C, T, nz):
    nb = B // NW
    nm = nb // CHUNK
    rows = CHUNK * C

    mesh = plsc.VectorSubcoreMesh(core_axis_name="c", subcore_axis_name="s")

    @pl.kernel(
        out_type=jax.ShapeDtypeStruct((NSLOT * B, T), jnp.float32),
        mesh=mesh,
        scratch_types=[
            pltpu.VMEM((rows, T), jnp.float32),
            pltpu.VMEM((rows, T), jnp.float32),
            pltpu.VMEM((rows, T), jnp.float32),
            pltpu.VMEM((rows, T), jnp.float32),
            pltpu.VMEM((rows, T), jnp.float32),
            pltpu.VMEM((nm, rows), jnp.int32),
            pltpu.VMEM((nz, rows), jnp.int32),
            pltpu.SemaphoreType.DMA,
            pltpu.SemaphoreType.DMA,
            pltpu.SemaphoreType.DMA,
            pltpu.SemaphoreType.DMA,
            pltpu.SemaphoreType.DMA,
            pltpu.SemaphoreType.DMA,
            pltpu.SemaphoreType.DMA,
            pltpu.SemaphoreType.DMA,
            pltpu.SemaphoreType.DMA,
        ],
    )
    def k(x_hbm, tab_d_hbm, tab_z_hbm, z_hbm, out_hbm,
          buf0, buf1, buf2, buf3, zbuf, didx, zidx,
          semL0, semL1, semL2, semL3,
          semD0, semD1, semD2, semD3, semZ):
        wid = lax.axis_index("s") * 2 + lax.axis_index("c")
        b_lo = wid * nb
        x_lo = b_lo * C

        bufs = (buf0, buf1, buf2, buf3)
        semL = (semL0, semL1, semL2, semL3)
        semD = (semD0, semD1, semD2, semD3)

        pltpu.sync_copy(tab_d_hbm.at[wid], didx)
        pltpu.sync_copy(tab_z_hbm.at[wid], zidx)
        pltpu.sync_copy(z_hbm, zbuf)

        def start_load(m, j):
            return pltpu.async_copy(
                x_hbm.at[pl.ds(x_lo + m * rows, rows)], bufs[j], semL[j])

        def start_dscatter(m, j):
            return pltpu.async_copy(bufs[j], out_hbm.at[didx.at[m]], semD[j])

        def start_zscatter(m):
            return pltpu.async_copy(zbuf, out_hbm.at[zidx.at[m]], semZ)

        def wait_load(j):
            pltpu.make_async_copy(
                x_hbm.at[pl.ds(0, rows)], bufs[j], semL[j]).wait()

        def wait_dscatter(j):
            pltpu.make_async_copy(
                bufs[j], out_hbm.at[didx.at[0]], semD[j]).wait()

        def wait_zscatter():
            pltpu.make_async_copy(zbuf, out_hbm.at[zidx.at[0]], semZ).wait()

        for m in range(NRING):
            start_load(m, m)
            start_zscatter(m)
            wait_load(m)
            start_dscatter(m, m)

        @pl.loop(NRING, nm, step=NRING)
        def _(g):
            for j in range(NRING):
                m = g + j
                wait_dscatter(j)
                start_load(m, j)

                @pl.when(m < nz)
                def _():
                    start_zscatter(m)

                @pl.when(m < nz + NRING)
                def _():
                    wait_zscatter()

                wait_load(j)
                start_dscatter(m, j)

        for j in range(NRING):
            wait_dscatter(j)

    return k(x_rows, tab_d, tab_z, zeros_src)


def kernel(x, channel_transformation):
    B, C, T = x.shape
    ct = channel_transformation.astype(jnp.int32)

    slots = jnp.arange(NSLOT, dtype=jnp.int32)
    occupied = (slots[:, None] == ct[None, :]).any(axis=1)
    pz = jnp.where(~occupied, size=NSLOT - C, fill_value=0)[0].astype(jnp.int32)

    rows = CHUNK * C
    nb = B // NW
    nm = nb // CHUNK
    nz = NSLOT - C
    ww = jnp.arange(NW, dtype=jnp.int32)[:, None, None]
    mm = jnp.arange(nm, dtype=jnp.int32)[None, :, None]
    ii = jnp.arange(rows, dtype=jnp.int32)[None, None, :]
    tab_d = ct[ii % C] * B + ww * nb + CHUNK * mm + ii // C
    kk = jnp.arange(nz, dtype=jnp.int32)[None, :, None]
    tab_z = pz[kk] * B + ww * nb + ii

    zeros_src = jnp.zeros((rows, T), jnp.float32)
    x_rows = x.reshape(B * C, T)

    out_rows = _sc_scatter(x_rows, tab_d.astype(jnp.int32),
                           tab_z.astype(jnp.int32), zeros_src, B, C, T, nz)
    return jnp.transpose(out_rows.reshape(GRID_H, GRID_W, B, T), (2, 0, 1, 3))
